# baseline (speedup 1.0000x reference)
import functools

import jax
import jax.numpy as jnp
from jax import lax
from jax.experimental import pallas as pl
from jax.experimental.pallas import tpu as pltpu
from jax.experimental.pallas import tpu_sc as plsc

VOCAB = 1000000
EMB = 64
HID = 128
B = 16384
L = 200

NC = 2
NS = 16
NW = NC * NS
BPW = B // NW
GE = 2
NCHUNK = BPW // GE
NVEC = EMB // 16


def _sc_pool_sums(ids_flat, table):
    mesh = plsc.VectorSubcoreMesh(core_axis_name="c", subcore_axis_name="s")

    @functools.partial(
        pl.kernel,
        mesh=mesh,
        compiler_params=pltpu.CompilerParams(use_tc_tiling_on_sc=False),
        out_type=jax.ShapeDtypeStruct((B, EMB), jnp.float32),
        scratch_types=[
            pltpu.VMEM((2, GE * L), jnp.int32),
            pltpu.VMEM((2, GE * L, EMB), jnp.float32),
            pltpu.VMEM((BPW, EMB), jnp.float32),
            pltpu.SemaphoreType.DMA((2,)),
        ],
    )
    def k(ids_hbm, table_hbm, out_hbm, ids_v, rows_v, out_v, gsem):
        wid = lax.axis_index("s") * NC + lax.axis_index("c")
        base = wid * BPW

        def issue(buf, chunk):
            pltpu.sync_copy(ids_hbm.at[pl.ds((base + chunk * GE) * L, GE * L)],
                            ids_v.at[buf])
            for e in range(GE):
                pltpu.make_async_copy(
                    table_hbm.at[ids_v.at[buf, pl.ds(e * L, L)]],
                    rows_v.at[buf, pl.ds(e * L, L)],
                    gsem.at[buf],
                ).start()

        def consume(buf, chunk):
            for e in range(GE):
                pltpu.make_async_copy(
                    table_hbm.at[ids_v.at[buf, pl.ds(e * L, L)]],
                    rows_v.at[buf, pl.ds(e * L, L)],
                    gsem.at[buf],
                ).wait()
            for e in range(GE):
                def body(i, acc):
                    return tuple(
                        acc[j] + rows_v[buf, e * L + i, pl.ds(j * 16, 16)]
                        for j in range(NVEC)
                    )
                acc = lax.fori_loop(
                    0, L, body,
                    tuple(jnp.zeros((16,), jnp.float32) for _ in range(NVEC)),
                    unroll=2,
                )
                for j in range(NVEC):
                    out_v[chunk * GE + e, pl.ds(j * 16, 16)] = acc[j]

        issue(0, 0)
        issue(1, 1)

        @pl.loop(0, (NCHUNK - 2) // 2)
        def _(it):
            g = it * 2
            for b in range(2):
                consume(b, g + b)
                issue(b, g + b + 2)

        consume(0, NCHUNK - 2)
        consume(1, NCHUNK - 1)

        pltpu.sync_copy(out_v, out_hbm.at[pl.ds(base, BPW)])

    return k(ids_flat, table)


BLK = 512


def _mlp_body(sums_ref, ids_ref, w1_ref, b1_ref, w2_ref, b2_ref, out_ref):
    ids = ids_ref[...]
    cnt = jnp.sum((ids > 0).astype(jnp.float32), axis=1, keepdims=True)
    pooled = sums_ref[...] / (cnt + 1e-9)
    h = jnp.maximum(
        jax.lax.dot_general(
            pooled, w1_ref[...], (((1,), (0,)), ((), ())),
            preferred_element_type=jnp.float32,
            precision=jax.lax.Precision.HIGHEST,
        ) + b1_ref[...],
        0.0,
    )
    out = jax.lax.dot_general(
        h, w2_ref[...], (((1,), (0,)), ((), ())),
        preferred_element_type=jnp.float32,
        precision=jax.lax.Precision.HIGHEST,
    ) + b2_ref[...]
    norm = jnp.maximum(
        jnp.sqrt(jnp.sum(out * out, axis=1, keepdims=True)), 1e-12)
    out_ref[...] = out / norm


def _tc_mlp(sums, ids, W1, b1, W2, b2):
    grid = (B // BLK,)
    return pl.pallas_call(
        _mlp_body,
        grid=grid,
        in_specs=[
            pl.BlockSpec((BLK, EMB), lambda i: (i, 0)),
            pl.BlockSpec((BLK, L), lambda i: (i, 0)),
            pl.BlockSpec((EMB, HID), lambda i: (0, 0)),
            pl.BlockSpec((1, HID), lambda i: (0, 0)),
            pl.BlockSpec((HID, HID), lambda i: (0, 0)),
            pl.BlockSpec((1, HID), lambda i: (0, 0)),
        ],
        out_specs=pl.BlockSpec((BLK, HID), lambda i: (i, 0)),
        out_shape=jax.ShapeDtypeStruct((B, HID), jnp.float32),
    )(sums, ids, W1, b1, W2, b2)


@jax.jit
def kernel(input_ids, table, W1, b1, W2, b2):
    ids = input_ids.astype(jnp.int32)
    ids_flat = jnp.reshape(ids, (B * L,))
    sums = _sc_pool_sums(ids_flat, table)
    return _tc_mlp(sums, ids, W1, b1.reshape(1, HID), W2, b2.reshape(1, HID))


# speedup vs baseline: 1.0043x; 1.0043x over previous
import functools

import jax
import jax.numpy as jnp
from jax import lax
from jax.experimental import pallas as pl
from jax.experimental.pallas import tpu as pltpu
from jax.experimental.pallas import tpu_sc as plsc

VOCAB = 1000000
EMB = 64
HID = 128
B = 16384
L = 200

NC = 2
NS = 16
NW = NC * NS
BPW = B // NW
GE = 2
NCHUNK = BPW // GE
NVEC = EMB // 16


RPB = 8
NRB = B // NW // RPB


def _sc_repack_ids(ids):
    mesh = plsc.VectorSubcoreMesh(core_axis_name="c", subcore_axis_name="s")

    @functools.partial(
        pl.kernel,
        mesh=mesh,
        compiler_params=pltpu.CompilerParams(use_tc_tiling_on_sc=True),
        out_type=jax.ShapeDtypeStruct((B * L,), jnp.int32),
        scratch_types=[
            pltpu.VMEM((RPB, 128), jnp.int32),
            pltpu.VMEM((RPB, 72), jnp.int32),
            pltpu.VMEM((RPB * L,), jnp.int32),
        ],
    )
    def k(ids_hbm, out_hbm, v0, v1, flat):
        wid = lax.axis_index("s") * NC + lax.axis_index("c")
        base = wid * (B // NW)

        @pl.loop(0, NRB)
        def _(blk):
            r = base + blk * RPB
            pltpu.sync_copy(ids_hbm.at[pl.ds(r, RPB), pl.ds(0, 128)], v0)
            pltpu.sync_copy(ids_hbm.at[pl.ds(r, RPB), pl.ds(128, 72)], v1)
            for i in range(RPB):
                for j in range(8):
                    flat[pl.ds(i * L + j * 16, 16)] = v0[i, pl.ds(j * 16, 16)]
                for j in range(4):
                    flat[pl.ds(i * L + 128 + j * 16, 16)] = v1[i, pl.ds(j * 16, 16)]
                flat[pl.ds(i * L + 184, 16)] = v1[i, pl.ds(56, 16)]
            pltpu.sync_copy(flat, out_hbm.at[pl.ds(r * L, RPB * L)])

    return k(ids)


def _sc_pool_sums(ids_flat, table):
    mesh = plsc.VectorSubcoreMesh(core_axis_name="c", subcore_axis_name="s")

    @functools.partial(
        pl.kernel,
        mesh=mesh,
        compiler_params=pltpu.CompilerParams(use_tc_tiling_on_sc=False),
        out_type=jax.ShapeDtypeStruct((B, EMB), jnp.float32),
        scratch_types=[
            pltpu.VMEM((2, GE * L), jnp.int32),
            pltpu.VMEM((2, GE * L, EMB), jnp.float32),
            pltpu.VMEM((BPW, EMB), jnp.float32),
            pltpu.SemaphoreType.DMA((2,)),
        ],
    )
    def k(ids_hbm, table_hbm, out_hbm, ids_v, rows_v, out_v, gsem):
        wid = lax.axis_index("s") * NC + lax.axis_index("c")
        base = wid * BPW

        def issue(buf, chunk):
            pltpu.sync_copy(ids_hbm.at[pl.ds((base + chunk * GE) * L, GE * L)],
                            ids_v.at[buf])
            for e in range(GE):
                pltpu.make_async_copy(
                    table_hbm.at[ids_v.at[buf, pl.ds(e * L, L)]],
                    rows_v.at[buf, pl.ds(e * L, L)],
                    gsem.at[buf],
                ).start()

        def consume(buf, chunk):
            for e in range(GE):
                pltpu.make_async_copy(
                    table_hbm.at[ids_v.at[buf, pl.ds(e * L, L)]],
                    rows_v.at[buf, pl.ds(e * L, L)],
                    gsem.at[buf],
                ).wait()
            for e in range(GE):
                def body(i, acc):
                    return tuple(
                        acc[j] + rows_v[buf, e * L + i, pl.ds(j * 16, 16)]
                        for j in range(NVEC)
                    )
                acc = lax.fori_loop(
                    0, L, body,
                    tuple(jnp.zeros((16,), jnp.float32) for _ in range(NVEC)),
                    unroll=2,
                )
                for j in range(NVEC):
                    out_v[chunk * GE + e, pl.ds(j * 16, 16)] = acc[j]

        issue(0, 0)
        issue(1, 1)

        @pl.loop(0, (NCHUNK - 2) // 2)
        def _(it):
            g = it * 2
            for b in range(2):
                consume(b, g + b)
                issue(b, g + b + 2)

        consume(0, NCHUNK - 2)
        consume(1, NCHUNK - 1)

        pltpu.sync_copy(out_v, out_hbm.at[pl.ds(base, BPW)])

    return k(ids_flat, table)


BLK = 512


def _mlp_body(sums_ref, ids_ref, w1_ref, b1_ref, w2_ref, b2_ref, out_ref):
    ids = ids_ref[...]
    cnt = jnp.sum((ids > 0).astype(jnp.float32), axis=1, keepdims=True)
    pooled = sums_ref[...] / (cnt + 1e-9)
    h = jnp.maximum(
        jax.lax.dot_general(
            pooled, w1_ref[...], (((1,), (0,)), ((), ())),
            preferred_element_type=jnp.float32,
            precision=jax.lax.Precision.HIGHEST,
        ) + b1_ref[...],
        0.0,
    )
    out = jax.lax.dot_general(
        h, w2_ref[...], (((1,), (0,)), ((), ())),
        preferred_element_type=jnp.float32,
        precision=jax.lax.Precision.HIGHEST,
    ) + b2_ref[...]
    norm = jnp.maximum(
        jnp.sqrt(jnp.sum(out * out, axis=1, keepdims=True)), 1e-12)
    out_ref[...] = out / norm


def _tc_mlp(sums, ids, W1, b1, W2, b2):
    grid = (B // BLK,)
    return pl.pallas_call(
        _mlp_body,
        grid=grid,
        in_specs=[
            pl.BlockSpec((BLK, EMB), lambda i: (i, 0)),
            pl.BlockSpec((BLK, L), lambda i: (i, 0)),
            pl.BlockSpec((EMB, HID), lambda i: (0, 0)),
            pl.BlockSpec((1, HID), lambda i: (0, 0)),
            pl.BlockSpec((HID, HID), lambda i: (0, 0)),
            pl.BlockSpec((1, HID), lambda i: (0, 0)),
        ],
        out_specs=pl.BlockSpec((BLK, HID), lambda i: (i, 0)),
        out_shape=jax.ShapeDtypeStruct((B, HID), jnp.float32),
    )(sums, ids, W1, b1, W2, b2)


@jax.jit
def kernel(input_ids, table, W1, b1, W2, b2):
    ids = input_ids.astype(jnp.int32)
    ids_flat = _sc_repack_ids(ids)
    sums = _sc_pool_sums(ids_flat, table)
    return _tc_mlp(sums, ids, W1, b1.reshape(1, HID), W2, b2.reshape(1, HID))
